# baseline (speedup 1.0000x reference)
import dataclasses
import functools

import jax
import jax.numpy as jnp
from jax.experimental import pallas as pl
from jax.experimental.pallas import tpu as pltpu
from jax.experimental.pallas import tpu_sc as plsc

CF = 2.0
TOPK = 2


def _sc_compiler_params():
    cp = pltpu.CompilerParams()
    if "needs_layout_passes" in pltpu.CompilerParams.__dataclass_fields__:
        cp = dataclasses.replace(cp, needs_layout_passes=False)
    return cp


def _gelu(x):
    return 0.5 * x * (1.0 + jax.lax.erf(x * 0.7071067811865476))


def _router_body(cap, E, x_ref, rw_ref, p1_ref, p2_ref, g1_ref, g2_ref,
                 loss_ref):
    T, D = x_ref.shape
    logits = jnp.dot(x_ref[...], rw_ref[...], preferred_element_type=jnp.float32)
    m = jnp.max(logits, axis=-1, keepdims=True)
    ex = jnp.exp(logits - m)
    probs = ex / jnp.sum(ex, axis=-1, keepdims=True)
    eids = jax.lax.broadcasted_iota(jnp.int32, (T, E), 1)
    i1 = jnp.min(jnp.where(logits == m, eids, E), axis=-1, keepdims=True)
    l2 = jnp.where(eids == i1, -jnp.inf, logits)
    m2 = jnp.max(l2, axis=-1, keepdims=True)
    i2 = jnp.min(jnp.where(l2 == m2, eids, E), axis=-1, keepdims=True)
    p1 = jnp.sum(jnp.where(eids == i1, probs, 0.0), axis=-1, keepdims=True)
    p2 = jnp.sum(jnp.where(eids == i2, probs, 0.0), axis=-1, keepdims=True)
    s = p1 + p2
    g1 = p1 / s
    g2 = p2 / s
    assigned = (eids == i1) | (eids == i2)
    af = assigned.astype(jnp.float32)
    r_io = jax.lax.broadcasted_iota(jnp.int32, (T, T), 0)
    c_io = jax.lax.broadcasted_iota(jnp.int32, (T, T), 1)
    tri = (c_io <= r_io).astype(jnp.float32)
    cum = jnp.dot(tri, af, preferred_element_type=jnp.float32)
    slot = cum - 1.0
    slot1 = jnp.sum(jnp.where(eids == i1, slot, 0.0), axis=-1, keepdims=True).astype(jnp.int32)
    slot2 = jnp.sum(jnp.where(eids == i2, slot, 0.0), axis=-1, keepdims=True).astype(jnp.int32)
    k1 = slot1 < cap
    k2 = slot2 < cap
    EC = E * cap
    p1_ref[...] = jnp.where(k1, i1 * cap + slot1, EC)
    p2_ref[...] = jnp.where(k2, i2 * cap + slot2, EC)
    g1_ref[...] = jnp.where(k1, g1, 0.0)
    g2_ref[...] = jnp.where(k2, g2, 0.0)
    usage = jnp.sum(af, axis=0, keepdims=True) / (T * TOPK)
    mpe = jnp.mean(probs, axis=0, keepdims=True)
    loss_ref[...] = jnp.sum(mpe * usage, axis=1, keepdims=True) * E


def _router_call(x2d, rw_t, cap, E):
    T, D = x2d.shape
    return pl.pallas_call(
        functools.partial(_router_body, cap, E),
        out_shape=(
            jax.ShapeDtypeStruct((T, 1), jnp.int32),
            jax.ShapeDtypeStruct((T, 1), jnp.int32),
            jax.ShapeDtypeStruct((T, 1), jnp.float32),
            jax.ShapeDtypeStruct((T, 1), jnp.float32),
            jax.ShapeDtypeStruct((1, 1), jnp.float32),
        ),
    )(x2d, rw_t)


def _ffn_body(xe_ref, w1_ref, b1_ref, w2_ref, b2_ref, ye_ref):
    j = pl.program_id(1)
    cap = xe_ref.shape[0]
    MT = 256
    w1b = w1_ref[0].astype(jnp.bfloat16)
    w2b = w2_ref[0].astype(jnp.bfloat16)
    b1v = b1_ref[0]
    b2v = b2_ref[0]
    for m in range(cap // MT):
        rs = pl.ds(m * MT, MT)
        xb = xe_ref[rs, :].astype(jnp.bfloat16)
        h = _gelu(jnp.dot(xb, w1b, preferred_element_type=jnp.float32) + b1v)
        contrib = jnp.dot(h.astype(jnp.bfloat16), w2b,
                          preferred_element_type=jnp.float32)

        @pl.when(j == 0)
        def _(contrib=contrib, rs=rs):
            ye_ref[rs, :] = contrib + b2v

        @pl.when(j != 0)
        def _(contrib=contrib, rs=rs):
            ye_ref[rs, :] = ye_ref[rs, :] + contrib


def _ffn_call(xe, w1, b1, w2, b2, cap, Hb=1024):
    _, D = xe.shape
    E, _, H = w1.shape
    EC = E * cap
    J = H // Hb
    return pl.pallas_call(
        _ffn_body,
        grid=(E, J),
        in_specs=[
            pl.BlockSpec((cap, D), lambda e, j: (e, 0)),
            pl.BlockSpec((1, D, Hb), lambda e, j: (e, 0, j)),
            pl.BlockSpec((1, 1, Hb), lambda e, j: (e, 0, j)),
            pl.BlockSpec((1, Hb, D), lambda e, j: (e, j, 0)),
            pl.BlockSpec((1, 1, D), lambda e, j: (e, 0, 0)),
        ],
        out_specs=pl.BlockSpec((cap, D), lambda e, j: (e, 0)),
        out_shape=jax.ShapeDtypeStruct((EC, D), jnp.float32),
    )(xe, w1, b1.reshape(E, 1, H), w2, b2.reshape(E, 1, D))


def _dispatch_call(x2d, p1f, p2f, xe_rows):
    T, D = x2d.shape
    mesh = plsc.VectorSubcoreMesh(core_axis_name="c", subcore_axis_name="s")
    NW = mesh.num_cores * mesh.num_subcores
    tok_per_w = T // NW
    CH = 16
    NCH = tok_per_w // CH

    @functools.partial(
        pl.kernel, mesh=mesh,
        out_type=jax.ShapeDtypeStruct((xe_rows, D), jnp.float32),
        compiler_params=_sc_compiler_params(),
        scratch_types=[
            pltpu.VMEM((tok_per_w,), jnp.int32),
            pltpu.VMEM((tok_per_w,), jnp.int32),
            pltpu.VMEM((CH, D), jnp.float32),
            pltpu.VMEM((CH, D), jnp.float32),
            pltpu.SemaphoreType.DMA,
            pltpu.SemaphoreType.DMA,
            pltpu.SemaphoreType.DMA,
            pltpu.SemaphoreType.DMA,
        ],
    )
    def dispatch_k(x_hbm, p1_hbm, p2_hbm, xe_hbm, pv1, pv2, buf0, buf1,
                   rs0, rs1, ws0, ws1):
        wid = jax.lax.axis_index("s") * 2 + jax.lax.axis_index("c")
        tbase = wid * tok_per_w
        pltpu.sync_copy(p1_hbm.at[pl.ds(tbase, tok_per_w)], pv1)
        pltpu.sync_copy(p2_hbm.at[pl.ds(tbase, tok_per_w)], pv2)
        bufs = (buf0, buf1)
        rsems = (rs0, rs1)
        wsems = (ws0, ws1)

        def start_read(c):
            b = c % 2
            return pltpu.async_copy(
                x_hbm.at[pl.ds(tbase + c * CH, CH)], bufs[b], rsems[b])

        rpend = {0: start_read(0)}
        wpend = {0: [], 1: []}
        for c in range(NCH):
            b = c % 2
            rpend[b].wait()
            i1 = pv1[pl.ds(c * CH, CH)]
            i2 = pv2[pl.ds(c * CH, CH)]
            wpend[b] = [
                pltpu.async_copy(bufs[b], xe_hbm.at[i1], wsems[b]),
                pltpu.async_copy(bufs[b], xe_hbm.at[i2], wsems[b]),
            ]
            if c + 1 < NCH:
                b2 = (c + 1) % 2
                for dsc in wpend[b2]:
                    dsc.wait()
                wpend[b2] = []
                rpend[b2] = start_read(c + 1)
        for lst in wpend.values():
            for dsc in lst:
                dsc.wait()

    return dispatch_k(x2d, p1f, p2f)


def _combine_call(ye, p1f, p2f, g1f, g2f, T, EC):
    _, D = ye.shape
    mesh = plsc.VectorSubcoreMesh(core_axis_name="c", subcore_axis_name="s")
    NW = mesh.num_cores * mesh.num_subcores
    tok_per_w = T // NW
    CH = 16
    NCH = tok_per_w // CH

    @functools.partial(
        pl.kernel, mesh=mesh,
        out_type=jax.ShapeDtypeStruct((T, D), jnp.float32),
        compiler_params=_sc_compiler_params(),
        scratch_types=[
            pltpu.VMEM((tok_per_w,), jnp.int32),
            pltpu.VMEM((tok_per_w,), jnp.int32),
            pltpu.VMEM((tok_per_w,), jnp.float32),
            pltpu.VMEM((tok_per_w,), jnp.float32),
            pltpu.VMEM((CH, D), jnp.float32),
            pltpu.VMEM((CH, D), jnp.float32),
            pltpu.VMEM((CH, D), jnp.float32),
            pltpu.SemaphoreType.DMA,
            pltpu.SemaphoreType.DMA,
        ],
    )
    def combine_k(ye_hbm, p1_hbm, p2_hbm, g1_hbm, g2_hbm, out_hbm,
                  pv1, pv2, gv1, gv2, bufa, bufb, ob, sema, semb):
        wid = jax.lax.axis_index("s") * 2 + jax.lax.axis_index("c")
        tbase = wid * tok_per_w
        pltpu.sync_copy(p1_hbm.at[pl.ds(tbase, tok_per_w)], pv1)
        pltpu.sync_copy(p2_hbm.at[pl.ds(tbase, tok_per_w)], pv2)
        pltpu.sync_copy(g1_hbm.at[pl.ds(tbase, tok_per_w)], gv1)
        pltpu.sync_copy(g2_hbm.at[pl.ds(tbase, tok_per_w)], gv2)
        lane = jax.lax.broadcasted_iota(jnp.int32, (16,), 0)

        def start(c):
            ia = jnp.minimum(pv1[pl.ds(c * CH, CH)], EC - 1)
            ib = jnp.minimum(pv2[pl.ds(c * CH, CH)], EC - 1)
            return (pltpu.async_copy(ye_hbm.at[ia], bufa, sema),
                    pltpu.async_copy(ye_hbm.at[ib], bufb, semb))

        pend = start(0)
        for c in range(NCH):
            for dsc in pend:
                dsc.wait()
            g1v = gv1[pl.ds(c * CH, CH)]
            g2v = gv2[pl.ds(c * CH, CH)]

            @pl.loop(0, CH)
            def _(i):
                ga = jnp.sum(jnp.where(lane == i, g1v, 0.0))
                gb = jnp.sum(jnp.where(lane == i, g2v, 0.0))

                @pl.loop(0, D // 16)
                def _(dd):
                    sl = pl.ds(dd * 16, 16)
                    ob[i, sl] = ga * bufa[i, sl] + gb * bufb[i, sl]

            pltpu.sync_copy(ob, out_hbm.at[pl.ds(tbase + c * CH, CH)])
            if c + 1 < NCH:
                pend = start(c + 1)

    return combine_k(ye, p1f, p2f, g1f, g2f)


def kernel(x, router_w, w1, b1, w2, b2, is_training):
    Bt, S, D = x.shape
    T = Bt * S
    E = router_w.shape[0]
    cap = int(T * CF * TOPK / E)
    EC = E * cap
    x2d = x.reshape(T, D)
    p1, p2, g1, g2, loss = _router_call(x2d, router_w.T, cap, E)
    p1f = p1.reshape(T)
    p2f = p2.reshape(T)
    g1f = g1.reshape(T)
    g2f = g2.reshape(T)
    xe = _dispatch_call(x2d, p1f, p2f, EC + cap)
    ye = _ffn_call(xe, w1, b1, w2, b2, cap)
    out = _combine_call(ye, p1f, p2f, g1f, g2f, T, EC)
    return out.reshape(Bt, S, D), loss.reshape(())


# speedup vs baseline: 1.1172x; 1.1172x over previous
import dataclasses
import functools

import jax
import jax.numpy as jnp
from jax.experimental import pallas as pl
from jax.experimental.pallas import tpu as pltpu
from jax.experimental.pallas import tpu_sc as plsc

CF = 2.0
TOPK = 2


def _sc_compiler_params():
    cp = pltpu.CompilerParams()
    if "needs_layout_passes" in pltpu.CompilerParams.__dataclass_fields__:
        cp = dataclasses.replace(cp, needs_layout_passes=False)
    return cp


def _gelu(x):
    return 0.5 * x * (1.0 + jax.lax.erf(x * 0.7071067811865476))


def _router_body(cap, E, x_ref, rw_ref, p1_ref, p2_ref, g1_ref, g2_ref,
                 loss_ref):
    T, D = x_ref.shape
    logits = jnp.dot(x_ref[...], rw_ref[...], preferred_element_type=jnp.float32)
    m = jnp.max(logits, axis=-1, keepdims=True)
    ex = jnp.exp(logits - m)
    probs = ex / jnp.sum(ex, axis=-1, keepdims=True)
    eids = jax.lax.broadcasted_iota(jnp.int32, (T, E), 1)
    i1 = jnp.min(jnp.where(logits == m, eids, E), axis=-1, keepdims=True)
    l2 = jnp.where(eids == i1, -jnp.inf, logits)
    m2 = jnp.max(l2, axis=-1, keepdims=True)
    i2 = jnp.min(jnp.where(l2 == m2, eids, E), axis=-1, keepdims=True)
    p1 = jnp.sum(jnp.where(eids == i1, probs, 0.0), axis=-1, keepdims=True)
    p2 = jnp.sum(jnp.where(eids == i2, probs, 0.0), axis=-1, keepdims=True)
    s = p1 + p2
    g1 = p1 / s
    g2 = p2 / s
    assigned = (eids == i1) | (eids == i2)
    af = assigned.astype(jnp.float32)
    r_io = jax.lax.broadcasted_iota(jnp.int32, (T, T), 0)
    c_io = jax.lax.broadcasted_iota(jnp.int32, (T, T), 1)
    tri = (c_io <= r_io).astype(jnp.float32)
    cum = jnp.dot(tri, af, preferred_element_type=jnp.float32)
    slot = cum - 1.0
    slot1 = jnp.sum(jnp.where(eids == i1, slot, 0.0), axis=-1, keepdims=True).astype(jnp.int32)
    slot2 = jnp.sum(jnp.where(eids == i2, slot, 0.0), axis=-1, keepdims=True).astype(jnp.int32)
    k1 = slot1 < cap
    k2 = slot2 < cap
    EC = E * cap
    p1_ref[...] = jnp.where(k1, i1 * cap + slot1, EC)
    p2_ref[...] = jnp.where(k2, i2 * cap + slot2, EC)
    g1_ref[...] = jnp.where(k1, g1, 0.0)
    g2_ref[...] = jnp.where(k2, g2, 0.0)
    usage = jnp.sum(af, axis=0, keepdims=True) / (T * TOPK)
    mpe = jnp.mean(probs, axis=0, keepdims=True)
    loss_ref[...] = jnp.sum(mpe * usage, axis=1, keepdims=True) * E


def _router_call(x2d, rw_t, cap, E):
    T, D = x2d.shape
    return pl.pallas_call(
        functools.partial(_router_body, cap, E),
        out_shape=(
            jax.ShapeDtypeStruct((T, 1), jnp.int32),
            jax.ShapeDtypeStruct((T, 1), jnp.int32),
            jax.ShapeDtypeStruct((T, 1), jnp.float32),
            jax.ShapeDtypeStruct((T, 1), jnp.float32),
            jax.ShapeDtypeStruct((1, 1), jnp.float32),
        ),
    )(x2d, rw_t)


def _ffn_body(xe_ref, w1_ref, b1_ref, w2_ref, b2_ref, ye_ref):
    j = pl.program_id(1)
    xb = xe_ref[...].astype(jnp.bfloat16)
    w1b = w1_ref[0].astype(jnp.bfloat16)
    h = jnp.dot(xb, w1b, preferred_element_type=jnp.float32) + b1_ref[0]
    h = _gelu(h)
    contrib = jnp.dot(h.astype(jnp.bfloat16), w2_ref[0].astype(jnp.bfloat16),
                      preferred_element_type=jnp.float32)

    @pl.when(j == 0)
    def _():
        ye_ref[...] = contrib + b2_ref[0]

    @pl.when(j != 0)
    def _():
        ye_ref[...] = ye_ref[...] + contrib


def _ffn_call(xe, w1, b1, w2, b2, cap, Hb=1024):
    _, D = xe.shape
    E, _, H = w1.shape
    EC = E * cap
    J = H // Hb
    return pl.pallas_call(
        _ffn_body,
        grid=(E, J),
        in_specs=[
            pl.BlockSpec((cap, D), lambda e, j: (e, 0)),
            pl.BlockSpec((1, D, Hb), lambda e, j: (e, 0, j)),
            pl.BlockSpec((1, 1, Hb), lambda e, j: (e, 0, j)),
            pl.BlockSpec((1, Hb, D), lambda e, j: (e, j, 0)),
            pl.BlockSpec((1, 1, D), lambda e, j: (e, 0, 0)),
        ],
        out_specs=pl.BlockSpec((cap, D), lambda e, j: (e, 0)),
        out_shape=jax.ShapeDtypeStruct((EC, D), jnp.float32),
    )(xe, w1, b1.reshape(E, 1, H), w2, b2.reshape(E, 1, D))


def _dispatch_call(x2d, p1f, p2f, xe_rows):
    T, D = x2d.shape
    mesh = plsc.VectorSubcoreMesh(core_axis_name="c", subcore_axis_name="s")
    NW = mesh.num_cores * mesh.num_subcores
    tok_per_w = T // NW
    CH = 16
    NCH = tok_per_w // CH

    @functools.partial(
        pl.kernel, mesh=mesh,
        out_type=jax.ShapeDtypeStruct((xe_rows, D), jnp.float32),
        compiler_params=_sc_compiler_params(),
        scratch_types=[
            pltpu.VMEM((tok_per_w,), jnp.int32),
            pltpu.VMEM((tok_per_w,), jnp.int32),
            pltpu.VMEM((CH, D), jnp.float32),
            pltpu.VMEM((CH, D), jnp.float32),
            pltpu.SemaphoreType.DMA,
            pltpu.SemaphoreType.DMA,
            pltpu.SemaphoreType.DMA,
            pltpu.SemaphoreType.DMA,
        ],
    )
    def dispatch_k(x_hbm, p1_hbm, p2_hbm, xe_hbm, pv1, pv2, buf0, buf1,
                   rs0, rs1, ws0, ws1):
        wid = jax.lax.axis_index("s") * 2 + jax.lax.axis_index("c")
        tbase = wid * tok_per_w
        pltpu.sync_copy(p1_hbm.at[pl.ds(tbase, tok_per_w)], pv1)
        pltpu.sync_copy(p2_hbm.at[pl.ds(tbase, tok_per_w)], pv2)
        bufs = (buf0, buf1)
        rsems = (rs0, rs1)
        wsems = (ws0, ws1)

        def start_read(c):
            b = c % 2
            return pltpu.async_copy(
                x_hbm.at[pl.ds(tbase + c * CH, CH)], bufs[b], rsems[b])

        rpend = {0: start_read(0)}
        wpend = {0: [], 1: []}
        for c in range(NCH):
            b = c % 2
            rpend[b].wait()
            i1 = pv1[pl.ds(c * CH, CH)]
            i2 = pv2[pl.ds(c * CH, CH)]
            wpend[b] = [
                pltpu.async_copy(bufs[b], xe_hbm.at[i1], wsems[b]),
                pltpu.async_copy(bufs[b], xe_hbm.at[i2], wsems[b]),
            ]
            if c + 1 < NCH:
                b2 = (c + 1) % 2
                for dsc in wpend[b2]:
                    dsc.wait()
                wpend[b2] = []
                rpend[b2] = start_read(c + 1)
        for lst in wpend.values():
            for dsc in lst:
                dsc.wait()

    return dispatch_k(x2d, p1f, p2f)


def _combine_call(ye, p1f, p2f, g1f, g2f, T, EC):
    _, D = ye.shape
    mesh = plsc.VectorSubcoreMesh(core_axis_name="c", subcore_axis_name="s")
    NW = mesh.num_cores * mesh.num_subcores
    tok_per_w = T // NW
    CH = 16
    NCH = tok_per_w // CH

    @functools.partial(
        pl.kernel, mesh=mesh,
        out_type=jax.ShapeDtypeStruct((T, D), jnp.float32),
        compiler_params=_sc_compiler_params(),
        scratch_types=[
            pltpu.VMEM((tok_per_w,), jnp.int32),
            pltpu.VMEM((tok_per_w,), jnp.int32),
            pltpu.VMEM((tok_per_w,), jnp.float32),
            pltpu.VMEM((tok_per_w,), jnp.float32),
            pltpu.VMEM((CH, D), jnp.float32),
            pltpu.VMEM((CH, D), jnp.float32),
            pltpu.VMEM((CH, D), jnp.float32),
            pltpu.SemaphoreType.DMA,
            pltpu.SemaphoreType.DMA,
        ],
    )
    def combine_k(ye_hbm, p1_hbm, p2_hbm, g1_hbm, g2_hbm, out_hbm,
                  pv1, pv2, gv1, gv2, bufa, bufb, ob, sema, semb):
        wid = jax.lax.axis_index("s") * 2 + jax.lax.axis_index("c")
        tbase = wid * tok_per_w
        pltpu.sync_copy(p1_hbm.at[pl.ds(tbase, tok_per_w)], pv1)
        pltpu.sync_copy(p2_hbm.at[pl.ds(tbase, tok_per_w)], pv2)
        pltpu.sync_copy(g1_hbm.at[pl.ds(tbase, tok_per_w)], gv1)
        pltpu.sync_copy(g2_hbm.at[pl.ds(tbase, tok_per_w)], gv2)
        lane = jax.lax.broadcasted_iota(jnp.int32, (16,), 0)

        def start(c):
            ia = jnp.minimum(pv1[pl.ds(c * CH, CH)], EC - 1)
            ib = jnp.minimum(pv2[pl.ds(c * CH, CH)], EC - 1)
            return (pltpu.async_copy(ye_hbm.at[ia], bufa, sema),
                    pltpu.async_copy(ye_hbm.at[ib], bufb, semb))

        pend = start(0)
        for c in range(NCH):
            for dsc in pend:
                dsc.wait()
            g1v = gv1[pl.ds(c * CH, CH)]
            g2v = gv2[pl.ds(c * CH, CH)]

            @pl.loop(0, CH)
            def _(i):
                ga = jnp.sum(jnp.where(lane == i, g1v, 0.0))
                gb = jnp.sum(jnp.where(lane == i, g2v, 0.0))

                @pl.loop(0, D // 16)
                def _(dd):
                    sl = pl.ds(dd * 16, 16)
                    ob[i, sl] = ga * bufa[i, sl] + gb * bufb[i, sl]

            pltpu.sync_copy(ob, out_hbm.at[pl.ds(tbase + c * CH, CH)])
            if c + 1 < NCH:
                pend = start(c + 1)

    return combine_k(ye, p1f, p2f, g1f, g2f)


def kernel(x, router_w, w1, b1, w2, b2, is_training):
    Bt, S, D = x.shape
    T = Bt * S
    E = router_w.shape[0]
    cap = int(T * CF * TOPK / E)
    EC = E * cap
    x2d = x.reshape(T, D)
    p1, p2, g1, g2, loss = _router_call(x2d, router_w.T, cap, E)
    p1f = p1.reshape(T)
    p2f = p2.reshape(T)
    g1f = g1.reshape(T)
    g2f = g2.reshape(T)
    xe = _dispatch_call(x2d, p1f, p2f, EC + cap)
    ye = _ffn_call(xe, w1, b1, w2, b2, cap)
    out = _combine_call(ye, p1f, p2f, g1f, g2f, T, EC)
    return out.reshape(Bt, S, D), loss.reshape(())


# speedup vs baseline: 1.1608x; 1.0390x over previous
import dataclasses
import functools

import jax
import jax.numpy as jnp
from jax.experimental import pallas as pl
from jax.experimental.pallas import tpu as pltpu
from jax.experimental.pallas import tpu_sc as plsc

CF = 2.0
TOPK = 2


def _sc_compiler_params():
    cp = pltpu.CompilerParams()
    if "needs_layout_passes" in pltpu.CompilerParams.__dataclass_fields__:
        cp = dataclasses.replace(cp, needs_layout_passes=False)
    return cp


def _gelu(x):
    return 0.5 * x * (1.0 + jax.lax.erf(x * 0.7071067811865476))


def _router_body(cap, E, x_ref, rw_ref, p1_ref, p2_ref, g1_ref, g2_ref,
                 loss_ref):
    T, D = x_ref.shape
    logits = jax.lax.dot_general(
        x_ref[...], rw_ref[...], (((1,), (1,)), ((), ())),
        preferred_element_type=jnp.float32)
    m = jnp.max(logits, axis=-1, keepdims=True)
    ex = jnp.exp(logits - m)
    probs = ex / jnp.sum(ex, axis=-1, keepdims=True)
    eids = jax.lax.broadcasted_iota(jnp.int32, (T, E), 1)
    i1 = jnp.min(jnp.where(logits == m, eids, E), axis=-1, keepdims=True)
    l2 = jnp.where(eids == i1, -jnp.inf, logits)
    m2 = jnp.max(l2, axis=-1, keepdims=True)
    i2 = jnp.min(jnp.where(l2 == m2, eids, E), axis=-1, keepdims=True)
    p1 = jnp.sum(jnp.where(eids == i1, probs, 0.0), axis=-1, keepdims=True)
    p2 = jnp.sum(jnp.where(eids == i2, probs, 0.0), axis=-1, keepdims=True)
    s = p1 + p2
    g1 = p1 / s
    g2 = p2 / s
    assigned = (eids == i1) | (eids == i2)
    af = assigned.astype(jnp.float32)
    r_io = jax.lax.broadcasted_iota(jnp.int32, (T, T), 0)
    c_io = jax.lax.broadcasted_iota(jnp.int32, (T, T), 1)
    tri = (c_io <= r_io).astype(jnp.float32)
    cum = jnp.dot(tri, af, preferred_element_type=jnp.float32)
    slot = cum - 1.0
    slot1 = jnp.sum(jnp.where(eids == i1, slot, 0.0), axis=-1, keepdims=True).astype(jnp.int32)
    slot2 = jnp.sum(jnp.where(eids == i2, slot, 0.0), axis=-1, keepdims=True).astype(jnp.int32)
    k1 = slot1 < cap
    k2 = slot2 < cap
    EC = E * cap
    p1_ref[...] = jnp.where(k1, i1 * cap + slot1, EC)
    p2_ref[...] = jnp.where(k2, i2 * cap + slot2, EC)
    g1_ref[...] = jnp.where(k1, g1, 0.0)
    g2_ref[...] = jnp.where(k2, g2, 0.0)
    usage = jnp.sum(af, axis=0, keepdims=True) / (T * TOPK)
    mpe = jnp.mean(probs, axis=0, keepdims=True)
    loss_ref[...] = jnp.sum(mpe * usage, axis=1, keepdims=True) * E


def _router_call(x2d, rw_t, cap, E):
    T, D = x2d.shape
    return pl.pallas_call(
        functools.partial(_router_body, cap, E),
        out_shape=(
            jax.ShapeDtypeStruct((T, 1), jnp.int32),
            jax.ShapeDtypeStruct((T, 1), jnp.int32),
            jax.ShapeDtypeStruct((T, 1), jnp.float32),
            jax.ShapeDtypeStruct((T, 1), jnp.float32),
            jax.ShapeDtypeStruct((1, 1), jnp.float32),
        ),
    )(x2d, rw_t)


def _ffn_body(xe_ref, w1_ref, b1_ref, w2_ref, b2_ref, ye_ref):
    j = pl.program_id(1)
    xb = xe_ref[...].astype(jnp.bfloat16)
    w1b = w1_ref[0].astype(jnp.bfloat16)
    h = jnp.dot(xb, w1b, preferred_element_type=jnp.float32) + b1_ref[0]
    h = _gelu(h)
    contrib = jnp.dot(h.astype(jnp.bfloat16), w2_ref[0].astype(jnp.bfloat16),
                      preferred_element_type=jnp.float32)

    @pl.when(j == 0)
    def _():
        ye_ref[...] = contrib + b2_ref[0]

    @pl.when(j != 0)
    def _():
        ye_ref[...] = ye_ref[...] + contrib


def _ffn_call(xe, w1, b1, w2, b2, cap, Hb=1024):
    _, D = xe.shape
    E, _, H = w1.shape
    EC = E * cap
    J = H // Hb
    return pl.pallas_call(
        _ffn_body,
        grid=(E, J),
        in_specs=[
            pl.BlockSpec((cap, D), lambda e, j: (e, 0)),
            pl.BlockSpec((1, D, Hb), lambda e, j: (e, 0, j)),
            pl.BlockSpec((1, 1, Hb), lambda e, j: (e, 0, j)),
            pl.BlockSpec((1, Hb, D), lambda e, j: (e, j, 0)),
            pl.BlockSpec((1, 1, D), lambda e, j: (e, 0, 0)),
        ],
        out_specs=pl.BlockSpec((cap, D), lambda e, j: (e, 0)),
        out_shape=jax.ShapeDtypeStruct((EC, D), jnp.float32),
    )(xe, w1, b1.reshape(E, 1, H), w2, b2.reshape(E, 1, D))


def _dispatch_call(x2d, p1f, p2f, xe_rows):
    T, D = x2d.shape
    mesh = plsc.VectorSubcoreMesh(core_axis_name="c", subcore_axis_name="s")
    NW = mesh.num_cores * mesh.num_subcores
    tok_per_w = T // NW
    CH = 16
    NCH = tok_per_w // CH

    @functools.partial(
        pl.kernel, mesh=mesh,
        out_type=jax.ShapeDtypeStruct((xe_rows, D), jnp.float32),
        compiler_params=_sc_compiler_params(),
        scratch_types=[
            pltpu.VMEM((tok_per_w,), jnp.int32),
            pltpu.VMEM((tok_per_w,), jnp.int32),
            pltpu.VMEM((CH, D), jnp.float32),
            pltpu.VMEM((CH, D), jnp.float32),
            pltpu.SemaphoreType.DMA,
            pltpu.SemaphoreType.DMA,
            pltpu.SemaphoreType.DMA,
            pltpu.SemaphoreType.DMA,
        ],
    )
    def dispatch_k(x_hbm, p1_hbm, p2_hbm, xe_hbm, pv1, pv2, buf0, buf1,
                   rs0, rs1, ws0, ws1):
        wid = jax.lax.axis_index("s") * 2 + jax.lax.axis_index("c")
        tbase = wid * tok_per_w
        pltpu.sync_copy(p1_hbm.at[pl.ds(tbase, tok_per_w)], pv1)
        pltpu.sync_copy(p2_hbm.at[pl.ds(tbase, tok_per_w)], pv2)
        bufs = (buf0, buf1)
        rsems = (rs0, rs1)
        wsems = (ws0, ws1)

        def start_read(c):
            b = c % 2
            return pltpu.async_copy(
                x_hbm.at[pl.ds(tbase + c * CH, CH)], bufs[b], rsems[b])

        rpend = {0: start_read(0)}
        wpend = {0: [], 1: []}
        for c in range(NCH):
            b = c % 2
            rpend[b].wait()
            i1 = pv1[pl.ds(c * CH, CH)]
            i2 = pv2[pl.ds(c * CH, CH)]
            wpend[b] = [
                pltpu.async_copy(bufs[b], xe_hbm.at[i1], wsems[b]),
                pltpu.async_copy(bufs[b], xe_hbm.at[i2], wsems[b]),
            ]
            if c + 1 < NCH:
                b2 = (c + 1) % 2
                for dsc in wpend[b2]:
                    dsc.wait()
                wpend[b2] = []
                rpend[b2] = start_read(c + 1)
        for lst in wpend.values():
            for dsc in lst:
                dsc.wait()

    return dispatch_k(x2d, p1f, p2f)


def _combine_call(ye, p1f, p2f, g1f, g2f, T, EC):
    _, D = ye.shape
    mesh = plsc.VectorSubcoreMesh(core_axis_name="c", subcore_axis_name="s")
    NW = mesh.num_cores * mesh.num_subcores
    tok_per_w = T // NW
    CH = 16
    NCH = tok_per_w // CH

    @functools.partial(
        pl.kernel, mesh=mesh,
        out_type=jax.ShapeDtypeStruct((T, D), jnp.float32),
        compiler_params=_sc_compiler_params(),
        scratch_types=[
            pltpu.VMEM((tok_per_w,), jnp.int32),
            pltpu.VMEM((tok_per_w,), jnp.int32),
            pltpu.VMEM((tok_per_w,), jnp.float32),
            pltpu.VMEM((tok_per_w,), jnp.float32),
            pltpu.VMEM((CH, D), jnp.float32),
            pltpu.VMEM((CH, D), jnp.float32),
            pltpu.VMEM((CH, D), jnp.float32),
            pltpu.VMEM((CH, D), jnp.float32),
            pltpu.VMEM((CH, D), jnp.float32),
            pltpu.VMEM((CH, D), jnp.float32),
            pltpu.SemaphoreType.DMA,
            pltpu.SemaphoreType.DMA,
            pltpu.SemaphoreType.DMA,
            pltpu.SemaphoreType.DMA,
            pltpu.SemaphoreType.DMA,
            pltpu.SemaphoreType.DMA,
        ],
    )
    def combine_k(ye_hbm, p1_hbm, p2_hbm, g1_hbm, g2_hbm, out_hbm,
                  pv1, pv2, gv1, gv2, bufa0, bufb0, bufa1, bufb1, ob0, ob1,
                  sa0, sb0, sa1, sb1, so0, so1):
        wid = jax.lax.axis_index("s") * 2 + jax.lax.axis_index("c")
        tbase = wid * tok_per_w
        pltpu.sync_copy(p1_hbm.at[pl.ds(tbase, tok_per_w)], pv1)
        pltpu.sync_copy(p2_hbm.at[pl.ds(tbase, tok_per_w)], pv2)
        pltpu.sync_copy(g1_hbm.at[pl.ds(tbase, tok_per_w)], gv1)
        pltpu.sync_copy(g2_hbm.at[pl.ds(tbase, tok_per_w)], gv2)
        lane = jax.lax.broadcasted_iota(jnp.int32, (16,), 0)
        bufa = (bufa0, bufa1)
        bufb = (bufb0, bufb1)
        obs = (ob0, ob1)
        sas = (sa0, sa1)
        sbs = (sb0, sb1)
        sos = (so0, so1)

        def start(c):
            b = c % 2
            ia = jnp.minimum(pv1[pl.ds(c * CH, CH)], EC - 1)
            ib = jnp.minimum(pv2[pl.ds(c * CH, CH)], EC - 1)
            return (pltpu.async_copy(ye_hbm.at[ia], bufa[b], sas[b]),
                    pltpu.async_copy(ye_hbm.at[ib], bufb[b], sbs[b]))

        pend = {0: start(0)}
        wpend = {0: None, 1: None}
        for c in range(NCH):
            b = c % 2
            if c + 1 < NCH:
                pend[(c + 1) % 2] = start(c + 1)
            for dsc in pend[b]:
                dsc.wait()
            g1v = gv1[pl.ds(c * CH, CH)]
            g2v = gv2[pl.ds(c * CH, CH)]
            if wpend[b] is not None:
                wpend[b].wait()
            ba, bb, ob = bufa[b], bufb[b], obs[b]

            @pl.loop(0, CH)
            def _(i, ba=ba, bb=bb, ob=ob, g1v=g1v, g2v=g2v):
                ga = jnp.sum(jnp.where(lane == i, g1v, 0.0))
                gb = jnp.sum(jnp.where(lane == i, g2v, 0.0))

                @pl.loop(0, D // 16)
                def _(dd):
                    sl = pl.ds(dd * 16, 16)
                    ob[i, sl] = ga * ba[i, sl] + gb * bb[i, sl]

            wpend[b] = pltpu.async_copy(
                obs[b], out_hbm.at[pl.ds(tbase + c * CH, CH)], sos[b])
        for w in wpend.values():
            if w is not None:
                w.wait()

    return combine_k(ye, p1f, p2f, g1f, g2f)


def kernel(x, router_w, w1, b1, w2, b2, is_training):
    Bt, S, D = x.shape
    T = Bt * S
    E = router_w.shape[0]
    cap = int(T * CF * TOPK / E)
    EC = E * cap
    x2d = x.reshape(T, D)
    p1, p2, g1, g2, loss = _router_call(x2d, router_w, cap, E)
    p1f = p1.reshape(T)
    p2f = p2.reshape(T)
    g1f = g1.reshape(T)
    g2f = g2.reshape(T)
    xe = _dispatch_call(x2d, p1f, p2f, EC + cap)
    ye = _ffn_call(xe, w1, b1, w2, b2, cap)
    out = _combine_call(ye, p1f, p2f, g1f, g2f, T, EC)
    return out.reshape(Bt, S, D), loss.reshape(())
